# baseline (speedup 1.0000x reference)
import functools

import jax
import jax.numpy as jnp
from jax import lax
from jax.experimental import pallas as pl
from jax.experimental.pallas import tpu as pltpu
from jax.experimental.pallas import tpu_sc as plsc

VOCAB = 1000000
D = 32
BATCH = 16384
HIST = 50
HP = 64

_info = plsc.get_sparse_core_info()
_NC, _NS = _info.num_cores, _info.num_subcores
_NW = _NC * _NS
_NB = BATCH // _NW
_IDX_PER_W = _NB * HP
_CHT = 32
_CHR = _CHT * HP
_NCH = _NB // _CHT

_mesh = plsc.VectorSubcoreMesh(core_axis_name="c", subcore_axis_name="s")


@functools.partial(
    pl.kernel,
    mesh=_mesh,
    out_type=jax.ShapeDtypeStruct((BATCH * HP, D), jnp.float32),
    scratch_types=[
        pltpu.VMEM((_IDX_PER_W,), jnp.int32),
        pltpu.VMEM((_CHR, D), jnp.float32),
        pltpu.SemaphoreType.DMA,
    ],
    compiler_params=pltpu.CompilerParams(
        use_tc_tiling_on_sc=False, needs_layout_passes=False),
)
def _gather_rows(idx_hbm, table_hbm, out_hbm, idx_v, buf_v, gsem):
    wid = lax.axis_index("s") * _NC + lax.axis_index("c")
    r0 = wid * _IDX_PER_W
    pltpu.sync_copy(idx_hbm.at[pl.ds(r0, _IDX_PER_W)], idx_v)

    def chunk(c, carry):
        pltpu.async_copy(
            table_hbm.at[idx_v.at[pl.ds(c * _CHR, _CHR)]], buf_v, gsem).wait()
        pltpu.sync_copy(buf_v, out_hbm.at[pl.ds(r0 + c * _CHR, _CHR)])
        return carry

    lax.fori_loop(0, _NCH, chunk, 0)


_CB = 256


def _transpose_body(x_ref, o_ref):
    x = x_ref[...]
    for s in range(HIST * D // 128):
        o_ref[pl.ds(s * 128, 128), :] = x[:, s, :].T
    rem = HIST * D % 128
    s = HIST * D // 128
    o_ref[pl.ds(s * 128, rem), :] = x[:, s, :rem].T


def _tc_transpose(o3):
    return pl.pallas_call(
        _transpose_body,
        grid=(BATCH // _CB,),
        in_specs=[pl.BlockSpec((_CB, HP // 4, 128), lambda i: (i, 0, 0))],
        out_specs=pl.BlockSpec((HIST * D, _CB), lambda i: (0, i)),
        out_shape=jax.ShapeDtypeStruct((HIST * D, BATCH), jnp.float32),
    )(o3)


def kernel(token_ids, w):
    idx = jnp.pad(token_ids, ((0, 0), (0, HP - HIST)))
    idx = idx.reshape(-1).astype(jnp.int32)
    out_lin = _gather_rows(idx, w)
    o3 = out_lin.reshape(BATCH, HP // 4, 128)
    out2d = _tc_transpose(o3)
    return out2d.reshape(HIST, D, BATCH).transpose(2, 0, 1)


# speedup vs baseline: 4.2532x; 4.2532x over previous
import functools

import jax
import jax.numpy as jnp
from jax import lax
from jax.experimental import pallas as pl
from jax.experimental.pallas import tpu as pltpu
from jax.experimental.pallas import tpu_sc as plsc

VOCAB = 1000000
D = 32
BATCH = 16384
HIST = 50
HP = 64

_info = plsc.get_sparse_core_info()
_NC, _NS = _info.num_cores, _info.num_subcores
_NW = _NC * _NS
_NB = BATCH // _NW
_IDX_PER_W = _NB * HP
_CHT = 32
_CHR = _CHT * HP
_NCH = _NB // _CHT

_mesh = plsc.VectorSubcoreMesh(core_axis_name="c", subcore_axis_name="s")


@functools.partial(
    pl.kernel,
    mesh=_mesh,
    out_type=jax.ShapeDtypeStruct((BATCH * HP, D), jnp.float32),
    scratch_types=[
        pltpu.VMEM((_IDX_PER_W,), jnp.int32),
        pltpu.VMEM((_CHR, D), jnp.float32),
        pltpu.SemaphoreType.DMA,
    ],
    compiler_params=pltpu.CompilerParams(
        use_tc_tiling_on_sc=False, needs_layout_passes=False),
)
def _gather_rows(idx_hbm, table_hbm, out_hbm, idx_v, buf_v, gsem):
    wid = lax.axis_index("s") * _NC + lax.axis_index("c")
    r0 = wid * _IDX_PER_W
    pltpu.sync_copy(idx_hbm.at[pl.ds(r0, _IDX_PER_W)], idx_v)

    def chunk(c, carry):
        pltpu.async_copy(
            table_hbm.at[idx_v.at[pl.ds(c * _CHR, _CHR)]], buf_v, gsem).wait()
        pltpu.sync_copy(buf_v, out_hbm.at[pl.ds(r0 + c * _CHR, _CHR)])
        return carry

    lax.fori_loop(0, _NCH, chunk, 0)


_CB = 256


def _transpose_body(x_ref, o_ref):
    x = x_ref[...]
    for s in range(HIST * D // 128):
        o_ref[pl.ds(s * 128, 128), :] = x[:, s, :].T
    rem = HIST * D % 128
    s = HIST * D // 128
    o_ref[pl.ds(s * 128, rem), :] = x[:, s, :rem].T


def _tc_transpose(o3):
    return pl.pallas_call(
        _transpose_body,
        grid=(BATCH // _CB,),
        in_specs=[pl.BlockSpec((_CB, HP // 4, 128), lambda i: (i, 0, 0))],
        out_specs=pl.BlockSpec((HIST * D, _CB), lambda i: (0, i)),
        out_shape=jax.ShapeDtypeStruct((HIST * D, BATCH), jnp.float32),
    )(o3)


def kernel(token_ids, w):
    idx = jnp.concatenate([token_ids, token_ids[:, : HP - HIST]], axis=1)
    idx = idx.reshape(-1).astype(jnp.int32)
    out_lin = _gather_rows(idx, w)
    o3 = out_lin.reshape(BATCH, HP // 4, 128)
    out2d = _tc_transpose(o3)
    return out2d.reshape(HIST, D, BATCH).transpose(2, 0, 1)


# speedup vs baseline: 4.6661x; 1.0971x over previous
import functools

import jax
import jax.numpy as jnp
from jax import lax
from jax.experimental import pallas as pl
from jax.experimental.pallas import tpu as pltpu
from jax.experimental.pallas import tpu_sc as plsc

VOCAB = 1000000
D = 32
BATCH = 16384
HIST = 50
HP = 64

_info = plsc.get_sparse_core_info()
_NC, _NS = _info.num_cores, _info.num_subcores
_NW = _NC * _NS
_NB = BATCH // _NW
_IDX_PER_W = _NB * HP
_CHT = 32
_CHR = _CHT * HP
_NCH = _NB // _CHT

_mesh = plsc.VectorSubcoreMesh(core_axis_name="c", subcore_axis_name="s")


@functools.partial(
    pl.kernel,
    mesh=_mesh,
    out_type=jax.ShapeDtypeStruct((BATCH * HP, D), jnp.float32),
    scratch_types=[
        pltpu.VMEM((_IDX_PER_W,), jnp.int32),
        pltpu.VMEM((_CHR, D), jnp.float32),
        pltpu.SemaphoreType.DMA,
    ],
    compiler_params=pltpu.CompilerParams(
        use_tc_tiling_on_sc=False, needs_layout_passes=False),
)
def _gather_rows(idx_hbm, table_hbm, out_hbm, idx_v, buf_v, gsem):
    wid = lax.axis_index("s") * _NC + lax.axis_index("c")
    r0 = wid * _IDX_PER_W
    pltpu.sync_copy(idx_hbm.at[pl.ds(r0, _IDX_PER_W)], idx_v)

    def chunk(c, carry):
        pltpu.async_copy(
            table_hbm.at[idx_v.at[pl.ds(c * _CHR, _CHR)]], buf_v, gsem).wait()
        pltpu.sync_copy(buf_v, out_hbm.at[pl.ds(r0 + c * _CHR, _CHR)])
        return carry

    lax.fori_loop(0, _NCH, chunk, 0)


_WB = 512
_WG = 489
_TROWS = 4 * _WG * _WB
_NWB = VOCAB // _WB + 1


def _wt_body(x0_ref, x1_ref, x2_ref, x3_ref, o_ref):
    o_ref[:, 0 * D:1 * D] = x0_ref[...].T
    o_ref[:, 1 * D:2 * D] = x1_ref[...].T
    o_ref[:, 2 * D:3 * D] = x2_ref[...].T
    o_ref[:, 3 * D:4 * D] = x3_ref[...].T


def _w_linearize(wT):
    specs = [
        pl.BlockSpec((D, _WB), functools.partial(
            lambda k, i: (0, jnp.minimum(4 * i + k, _NWB - 1)), k))
        for k in range(4)
    ]
    return pl.pallas_call(
        _wt_body,
        grid=(_WG,),
        in_specs=specs,
        out_specs=pl.BlockSpec((_WB, 128), lambda i: (i, 0)),
        out_shape=jax.ShapeDtypeStruct((_TROWS * D // 128, 128), jnp.float32),
    )(wT, wT, wT, wT)


_CB = 256


def _transpose_body(x_ref, o_ref):
    x = x_ref[...]
    for s in range(HIST * D // 128):
        o_ref[pl.ds(s * 128, 128), :] = x[:, s, :].T
    rem = HIST * D % 128
    s = HIST * D // 128
    o_ref[pl.ds(s * 128, rem), :] = x[:, s, :rem].T


def _tc_transpose(o3):
    return pl.pallas_call(
        _transpose_body,
        grid=(BATCH // _CB,),
        in_specs=[pl.BlockSpec((_CB, HP // 4, 128), lambda i: (i, 0, 0))],
        out_specs=pl.BlockSpec((HIST * D, _CB), lambda i: (0, i)),
        out_shape=jax.ShapeDtypeStruct((HIST * D, BATCH), jnp.float32),
    )(o3)


def kernel(token_ids, w):
    idx = jnp.concatenate([token_ids, token_ids[:, : HP - HIST]], axis=1)
    idx = idx.reshape(-1).astype(jnp.int32)
    idx = ((idx // 2048) * 2048 + 4 * (idx % 512)
           + (idx // 512) % 4)
    w_rows = _w_linearize(w.T).reshape(_TROWS, D)
    out_lin = _gather_rows(idx, w_rows)
    o3 = out_lin.reshape(BATCH, HP // 4, 128)
    out2d = _tc_transpose(o3)
    return out2d.reshape(HIST, D, BATCH).transpose(2, 0, 1)


# speedup vs baseline: 6.9632x; 1.4923x over previous
import functools

import jax
import jax.numpy as jnp
from jax import lax
from jax.experimental import pallas as pl
from jax.experimental.pallas import tpu as pltpu
from jax.experimental.pallas import tpu_sc as plsc

VOCAB = 1000000
D = 32
BATCH = 16384
HIST = 50
HP = 64

_info = plsc.get_sparse_core_info()
_NC, _NS = _info.num_cores, _info.num_subcores
_NW = _NC * _NS
_NB = BATCH // _NW
_IDX_PER_W = _NB * HP
_CHT = 32
_CHR = _CHT * HP
_NCH = _NB // _CHT

_mesh = plsc.VectorSubcoreMesh(core_axis_name="c", subcore_axis_name="s")


@functools.partial(
    pl.kernel,
    mesh=_mesh,
    out_type=jax.ShapeDtypeStruct((BATCH * HP, D), jnp.float32),
    scratch_types=[
        pltpu.VMEM((_IDX_PER_W,), jnp.int32),
        pltpu.VMEM((_CHR, D), jnp.float32),
        pltpu.SemaphoreType.DMA,
    ],
    compiler_params=pltpu.CompilerParams(
        use_tc_tiling_on_sc=False, needs_layout_passes=False),
)
def _gather_rows(idx_hbm, table_hbm, out_hbm, idx_v, buf_v, gsem):
    wid = lax.axis_index("s") * _NC + lax.axis_index("c")
    r0 = wid * _IDX_PER_W
    pltpu.sync_copy(idx_hbm.at[pl.ds(r0, _IDX_PER_W)], idx_v)

    def chunk(c, carry):
        pltpu.async_copy(
            table_hbm.at[idx_v.at[pl.ds(c * _CHR, _CHR)]], buf_v, gsem).wait()
        pltpu.sync_copy(buf_v, out_hbm.at[pl.ds(r0 + c * _CHR, _CHR)])
        return carry

    lax.fori_loop(0, _NCH, chunk, 0)


_WB = 1024
_WG = (VOCAB + 4 * _WB - 1) // (4 * _WB)
_TROWS = 4 * _WG * _WB
_NWB = VOCAB // _WB + 1


def _wt_body(x0_ref, x1_ref, x2_ref, x3_ref, o_ref):
    x = jnp.concatenate(
        [x0_ref[...], x1_ref[...], x2_ref[...], x3_ref[...]], axis=0)
    o_ref[...] = x.T


def _w_linearize(wT):
    specs = [
        pl.BlockSpec((D, _WB), functools.partial(
            lambda k, i: (0, jnp.minimum(4 * i + k, _NWB - 1)), k))
        for k in range(4)
    ]
    return pl.pallas_call(
        _wt_body,
        grid=(_WG,),
        in_specs=specs,
        out_specs=pl.BlockSpec((_WB, 128), lambda i: (i, 0)),
        out_shape=jax.ShapeDtypeStruct((_TROWS * D // 128, 128), jnp.float32),
    )(wT, wT, wT, wT)


_CB = 256


def _transpose_body(x_ref, o_ref):
    x = x_ref[...]
    for s in range(HIST * D // 128):
        o_ref[pl.ds(s * 128, 128), :] = x[:, s, :].T
    rem = HIST * D % 128
    s = HIST * D // 128
    o_ref[pl.ds(s * 128, rem), :] = x[:, s, :rem].T


def _tc_transpose(o3):
    return pl.pallas_call(
        _transpose_body,
        grid=(BATCH // _CB,),
        in_specs=[pl.BlockSpec((_CB, HP // 4, 128), lambda i: (i, 0, 0))],
        out_specs=pl.BlockSpec((HIST * D, _CB), lambda i: (0, i)),
        out_shape=jax.ShapeDtypeStruct((HIST * D, BATCH), jnp.float32),
    )(o3)


def kernel(token_ids, w):
    idx = jnp.concatenate([token_ids, token_ids[:, : HP - HIST]], axis=1)
    idx = idx.reshape(-1).astype(jnp.int32)
    idx = ((idx // (4 * _WB)) * (4 * _WB) + 4 * (idx % _WB)
           + (idx // _WB) % 4)
    w_rows = _w_linearize(w.T).reshape(_TROWS, D)
    out_lin = _gather_rows(idx, w_rows)
    o3 = out_lin.reshape(BATCH, HP // 4, 128)
    out2d = _tc_transpose(o3)
    return out2d.reshape(HIST, D, BATCH).transpose(2, 0, 1)


# speedup vs baseline: 8.3648x; 1.2013x over previous
import functools

import jax
import jax.numpy as jnp
from jax import lax
from jax.experimental import pallas as pl
from jax.experimental.pallas import tpu as pltpu
from jax.experimental.pallas import tpu_sc as plsc

VOCAB = 1000000
D = 32
BATCH = 16384
HIST = 50
HP = 64

_info = plsc.get_sparse_core_info()
_NC, _NS = _info.num_cores, _info.num_subcores
_NW = _NC * _NS
_NB = BATCH // _NW
_IDX_PER_W = _NB * HP
_CHT = 32
_CHR = _CHT * HP
_NCH = _NB // _CHT

_mesh = plsc.VectorSubcoreMesh(core_axis_name="c", subcore_axis_name="s")


@functools.partial(
    pl.kernel,
    mesh=_mesh,
    out_type=jax.ShapeDtypeStruct((BATCH * HP, D), jnp.float32),
    scratch_types=[
        pltpu.VMEM((_IDX_PER_W,), jnp.int32),
        pltpu.VMEM((_CHR, D), jnp.float32),
        pltpu.SemaphoreType.DMA,
    ],
    compiler_params=pltpu.CompilerParams(
        use_tc_tiling_on_sc=False, needs_layout_passes=False),
)
def _gather_rows(idx_hbm, table_hbm, out_hbm, idx_v, buf_v, gsem):
    wid = lax.axis_index("s") * _NC + lax.axis_index("c")
    r0 = wid * _IDX_PER_W
    pltpu.sync_copy(idx_hbm.at[pl.ds(r0, _IDX_PER_W)], idx_v)

    def chunk(c, carry):
        pltpu.async_copy(
            table_hbm.at[idx_v.at[pl.ds(c * _CHR, _CHR)]], buf_v, gsem).wait()
        pltpu.sync_copy(buf_v, out_hbm.at[pl.ds(r0 + c * _CHR, _CHR)])
        return carry

    lax.fori_loop(0, _NCH, chunk, 0)


_WB = 2048
_WG = (VOCAB + 4 * _WB - 1) // (4 * _WB)
_TROWS = 4 * _WG * _WB
_NWB = VOCAB // _WB + 1


def _wt_body(x0_ref, x1_ref, x2_ref, x3_ref, o_ref):
    x = jnp.concatenate(
        [x0_ref[...], x1_ref[...], x2_ref[...], x3_ref[...]], axis=0)
    o_ref[...] = x.T


def _w_linearize(wT):
    specs = [
        pl.BlockSpec((D, _WB), functools.partial(
            lambda k, i: (0, jnp.minimum(4 * i + k, _NWB - 1)), k))
        for k in range(4)
    ]
    return pl.pallas_call(
        _wt_body,
        grid=(_WG,),
        in_specs=specs,
        out_specs=pl.BlockSpec((_WB, 128), lambda i: (i, 0)),
        out_shape=jax.ShapeDtypeStruct((_TROWS * D // 128, 128), jnp.float32),
    )(wT, wT, wT, wT)


_CB = 512


def _transpose_body(x_ref, o_ref):
    x = x_ref[...]
    for s in range(HIST * D // 128):
        o_ref[pl.ds(s * 128, 128), :] = x[:, s, :].T
    rem = HIST * D % 128
    s = HIST * D // 128
    o_ref[pl.ds(s * 128, rem), :] = x[:, s, :rem].T


def _tc_transpose(o3):
    return pl.pallas_call(
        _transpose_body,
        grid=(BATCH // _CB,),
        in_specs=[pl.BlockSpec((_CB, HP // 4, 128), lambda i: (i, 0, 0))],
        out_specs=pl.BlockSpec((HIST * D, _CB), lambda i: (0, i)),
        out_shape=jax.ShapeDtypeStruct((HIST * D, BATCH), jnp.float32),
    )(o3)


def kernel(token_ids, w):
    idx = jnp.concatenate([token_ids, token_ids[:, : HP - HIST]], axis=1)
    idx = idx.reshape(-1).astype(jnp.int32)
    idx = ((idx // (4 * _WB)) * (4 * _WB) + 4 * (idx % _WB)
           + (idx // _WB) % 4)
    w_rows = _w_linearize(w.T).reshape(_TROWS, D)
    out_lin = _gather_rows(idx, w_rows)
    o3 = out_lin.reshape(BATCH, HP // 4, 128)
    out2d = _tc_transpose(o3)
    return out2d.reshape(HIST, D, BATCH).transpose(2, 0, 1)


# speedup vs baseline: 9.6241x; 1.1505x over previous
import functools

import jax
import jax.numpy as jnp
from jax import lax
from jax.experimental import pallas as pl
from jax.experimental.pallas import tpu as pltpu
from jax.experimental.pallas import tpu_sc as plsc

VOCAB = 1000000
D = 32
BATCH = 16384
HIST = 50
HP = 64

_info = plsc.get_sparse_core_info()
_NC, _NS = _info.num_cores, _info.num_subcores
_NW = _NC * _NS
_NB = BATCH // _NW
_IDX_PER_W = _NB * HP
_CHT = 32
_CHR = _CHT * HP
_NCH = _NB // _CHT

_mesh = plsc.VectorSubcoreMesh(core_axis_name="c", subcore_axis_name="s")


@functools.partial(
    pl.kernel,
    mesh=_mesh,
    out_type=jax.ShapeDtypeStruct((BATCH * HP, D), jnp.float32),
    scratch_types=[
        pltpu.VMEM((_IDX_PER_W,), jnp.int32),
        pltpu.VMEM((_CHR, D), jnp.float32),
        pltpu.SemaphoreType.DMA,
    ],
    compiler_params=pltpu.CompilerParams(
        use_tc_tiling_on_sc=False, needs_layout_passes=False),
)
def _gather_rows(idx_hbm, table_hbm, out_hbm, idx_v, buf_v, gsem):
    wid = lax.axis_index("s") * _NC + lax.axis_index("c")
    r0 = wid * _IDX_PER_W
    pltpu.sync_copy(idx_hbm.at[pl.ds(r0, _IDX_PER_W)], idx_v)

    def chunk(c, carry):
        pltpu.async_copy(
            table_hbm.at[idx_v.at[pl.ds(c * _CHR, _CHR)]], buf_v, gsem).wait()
        pltpu.sync_copy(buf_v, out_hbm.at[pl.ds(r0 + c * _CHR, _CHR)])
        return carry

    lax.fori_loop(0, _NCH, chunk, 0)


_WB = 4096
_WG = (VOCAB + 4 * _WB - 1) // (4 * _WB)
_TROWS = 4 * _WG * _WB
_NWB = VOCAB // _WB + 1


def _wt_body(x0_ref, x1_ref, x2_ref, x3_ref, o_ref):
    x = jnp.concatenate(
        [x0_ref[...], x1_ref[...], x2_ref[...], x3_ref[...]], axis=0)
    o_ref[...] = x.T


def _w_linearize(wT):
    specs = [
        pl.BlockSpec((D, _WB), functools.partial(
            lambda k, i: (0, jnp.minimum(4 * i + k, _NWB - 1)), k))
        for k in range(4)
    ]
    return pl.pallas_call(
        _wt_body,
        grid=(_WG,),
        in_specs=specs,
        out_specs=pl.BlockSpec((_WB, 128), lambda i: (i, 0)),
        out_shape=jax.ShapeDtypeStruct((_TROWS * D // 128, 128), jnp.float32),
    )(wT, wT, wT, wT)


_CB = 1024


def _transpose_body(x_ref, o_ref):
    x = x_ref[...]
    for s in range(HIST * D // 128):
        o_ref[pl.ds(s * 128, 128), :] = x[:, s, :].T
    rem = HIST * D % 128
    s = HIST * D // 128
    o_ref[pl.ds(s * 128, rem), :] = x[:, s, :rem].T


def _tc_transpose(o3):
    return pl.pallas_call(
        _transpose_body,
        grid=(BATCH // _CB,),
        in_specs=[pl.BlockSpec((_CB, HP // 4, 128), lambda i: (i, 0, 0))],
        out_specs=pl.BlockSpec((HIST * D, _CB), lambda i: (0, i)),
        out_shape=jax.ShapeDtypeStruct((HIST * D, BATCH), jnp.float32),
    )(o3)


def kernel(token_ids, w):
    idx = jnp.concatenate([token_ids, token_ids[:, : HP - HIST]], axis=1)
    idx = idx.reshape(-1).astype(jnp.int32)
    idx = ((idx // (4 * _WB)) * (4 * _WB) + 4 * (idx % _WB)
           + (idx // _WB) % 4)
    w_rows = _w_linearize(w.T).reshape(_TROWS, D)
    out_lin = _gather_rows(idx, w_rows)
    o3 = out_lin.reshape(BATCH, HP // 4, 128)
    out2d = _tc_transpose(o3)
    return out2d.reshape(HIST, D, BATCH).transpose(2, 0, 1)


# speedup vs baseline: 10.0719x; 1.0465x over previous
import functools

import jax
import jax.numpy as jnp
from jax import lax
from jax.experimental import pallas as pl
from jax.experimental.pallas import tpu as pltpu
from jax.experimental.pallas import tpu_sc as plsc

VOCAB = 1000000
D = 32
BATCH = 16384
HIST = 50
HP = 64

_info = plsc.get_sparse_core_info()
_NC, _NS = _info.num_cores, _info.num_subcores
_NW = _NC * _NS
_NB = BATCH // _NW
_IDX_PER_W = _NB * HP
_CHT = 32
_CHR = _CHT * HP
_NCH = _NB // _CHT

_mesh = plsc.VectorSubcoreMesh(core_axis_name="c", subcore_axis_name="s")


@functools.partial(
    pl.kernel,
    mesh=_mesh,
    out_type=jax.ShapeDtypeStruct((BATCH * HP, D), jnp.float32),
    scratch_types=[
        pltpu.VMEM((_IDX_PER_W,), jnp.int32),
        pltpu.VMEM((_CHR, D), jnp.float32),
        pltpu.SemaphoreType.DMA,
    ],
    compiler_params=pltpu.CompilerParams(
        use_tc_tiling_on_sc=False, needs_layout_passes=False),
)
def _gather_rows(idx_hbm, table_hbm, out_hbm, idx_v, buf_v, gsem):
    wid = lax.axis_index("s") * _NC + lax.axis_index("c")
    r0 = wid * _IDX_PER_W
    pltpu.sync_copy(idx_hbm.at[pl.ds(r0, _IDX_PER_W)], idx_v)

    def chunk(c, carry):
        pltpu.async_copy(
            table_hbm.at[idx_v.at[pl.ds(c * _CHR, _CHR)]], buf_v, gsem).wait()
        pltpu.sync_copy(buf_v, out_hbm.at[pl.ds(r0 + c * _CHR, _CHR)])
        return carry

    lax.fori_loop(0, _NCH, chunk, 0)


_WB = 8192
_WG = (VOCAB + 4 * _WB - 1) // (4 * _WB)
_TROWS = 4 * _WG * _WB
_NWB = VOCAB // _WB + 1


def _wt_body(x0_ref, x1_ref, x2_ref, x3_ref, o_ref):
    x = jnp.concatenate(
        [x0_ref[...], x1_ref[...], x2_ref[...], x3_ref[...]], axis=0)
    o_ref[...] = x.T


def _w_linearize(wT):
    specs = [
        pl.BlockSpec((D, _WB), functools.partial(
            lambda k, i: (0, jnp.minimum(4 * i + k, _NWB - 1)), k))
        for k in range(4)
    ]
    return pl.pallas_call(
        _wt_body,
        grid=(_WG,),
        in_specs=specs,
        out_specs=pl.BlockSpec((_WB, 128), lambda i: (i, 0)),
        out_shape=jax.ShapeDtypeStruct((_TROWS * D // 128, 128), jnp.float32),
    )(wT, wT, wT, wT)


_CB = 1024


def _transpose_body(x_ref, o_ref):
    x = x_ref[...]
    for s in range(HIST * D // 128):
        o_ref[pl.ds(s * 128, 128), :] = x[:, s, :].T
    rem = HIST * D % 128
    s = HIST * D // 128
    o_ref[pl.ds(s * 128, rem), :] = x[:, s, :rem].T


def _tc_transpose(o3):
    return pl.pallas_call(
        _transpose_body,
        grid=(BATCH // _CB,),
        in_specs=[pl.BlockSpec((_CB, HP // 4, 128), lambda i: (i, 0, 0))],
        out_specs=pl.BlockSpec((HIST * D, _CB), lambda i: (0, i)),
        out_shape=jax.ShapeDtypeStruct((HIST * D, BATCH), jnp.float32),
    )(o3)


def kernel(token_ids, w):
    idx = jnp.concatenate([token_ids, token_ids[:, : HP - HIST]], axis=1)
    idx = idx.reshape(-1).astype(jnp.int32)
    idx = ((idx // (4 * _WB)) * (4 * _WB) + 4 * (idx % _WB)
           + (idx // _WB) % 4)
    w_rows = _w_linearize(w.T).reshape(_TROWS, D)
    out_lin = _gather_rows(idx, w_rows)
    o3 = out_lin.reshape(BATCH, HP // 4, 128)
    out2d = _tc_transpose(o3)
    return out2d.reshape(HIST, D, BATCH).transpose(2, 0, 1)
